# baseline (speedup 1.0000x reference)
import functools

import jax
import jax.numpy as jnp
from jax import lax
from jax.experimental import pallas as pl
from jax.experimental.pallas import tpu as pltpu
from jax.experimental.pallas import tpu_sc as plsc

X = 512
XY = X * X
B = 4
C = 64
P = 24000
NSUB = 16
NCORE = 2
RANGE = XY // NSUB
WCAP = RANGE + 64
K = 64
NROW = C * K // 128
ZWORDS = 16384


def _make_kernel():
  mesh = plsc.VectorSubcoreMesh(core_axis_name="c", subcore_axis_name="s")

  @functools.partial(
      pl.kernel,
      out_type=jax.ShapeDtypeStruct((B * C * XY,), jnp.float32),
      mesh=mesh,
      compiler_params=pltpu.CompilerParams(
          needs_layout_passes=False, use_tc_tiling_on_sc=False),
      scratch_types=[
          pltpu.VMEM((P,), jnp.int32),
          pltpu.VMEM((P,), jnp.int32),
          pltpu.VMEM((RANGE,), jnp.int32),
          pltpu.VMEM((WCAP,), jnp.int32),
          pltpu.VMEM((WCAP,), jnp.int32),
          pltpu.VMEM((ZWORDS,), jnp.float32),
          pltpu.VMEM((K, C), jnp.float32),
          pltpu.VMEM((C * K,), jnp.float32),
          pltpu.VMEM((C * K,), jnp.int32),
          pltpu.SemaphoreType.DMA,
          pltpu.SemaphoreType.DMA,
          pltpu.SemaphoreType.DMA,
      ],
  )
  def scatter_kernel(feat_hbm, xs_hbm, ys_hbm, out_hbm,
                     xbuf, ybuf, cmap, wcell, wp, zbuf, rowbuf, colbuf, didx,
                     zsem, gsem, ssem):
    core = lax.axis_index("c")
    sub = lax.axis_index("s")
    lo = sub * RANGE
    iota = lax.iota(jnp.int32, 16)
    zero16 = jnp.zeros((16,), jnp.float32)
    minus1 = jnp.full((16,), -1, jnp.int32)
    widxs = [t * 16 + iota for t in range(K // 16)]

    def zb_body(i, carry):
      zbuf[pl.ds(i * 16, 16)] = zero16
      return carry
    lax.fori_loop(0, ZWORDS // 16, zb_body, 0)

    for bl in range(B // NCORE):
      b = core * (B // NCORE) + bl

      zbase = (b * C + 4 * sub) * XY
      zcopies = [
          pltpu.async_copy(
              zbuf, out_hbm.at[pl.ds(zbase + i * ZWORDS, ZWORDS)], zsem)
          for i in range(4 * XY // ZWORDS)
      ]

      pltpu.sync_copy(xs_hbm.at[pl.ds(b * P, P)], xbuf)
      pltpu.sync_copy(ys_hbm.at[pl.ds(b * P, P)], ybuf)

      def mi_body(i, carry):
        cmap[pl.ds(i * 16, 16)] = minus1
        return carry
      lax.fori_loop(0, RANGE // 16, mi_body, 0)

      pbase = b * P
      def scan_body(i, carry):
        xv = xbuf[pl.ds(i * 16, 16)]
        yv = ybuf[pl.ds(i * 16, 16)]
        rel = (yv * X + xv) - lo
        m = (rel >= 0) & (rel < RANGE)
        pv = (pbase + i * 16) + iota
        plsc.store_scatter(cmap, [rel], pv, mask=m)
        return carry
      lax.fori_loop(0, P // 16, scan_body, 0)

      def comp_body(j, cnt):
        mv = cmap[pl.ds(j * 16, 16)]
        m = mv >= 0
        cellv = (lo + j * 16) + iota
        plsc.store_compressed(wcell.at[pl.ds(cnt, 16)], cellv, mask=m)
        plsc.store_compressed(wp.at[pl.ds(cnt, 16)], mv, mask=m)
        return cnt + jnp.max(plsc.all_reduce_population_count(m))
      count = lax.fori_loop(0, RANGE // 16, comp_body, jnp.int32(0))
      nchunks = (count + (K - 1)) // K

      @pl.when(count > 0)
      def _():
        zidx = jnp.zeros((16,), jnp.int32)
        w0 = plsc.load_gather(wp, [zidx])
        c0 = plsc.load_gather(wcell, [zidx])
        base = (nchunks - 1) * K
        for t in range(K // 16):
          pos = base + t * 16 + iota
          m = pos >= count
          plsc.store_scatter(wp, [pos], w0, mask=m)
          plsc.store_scatter(wcell, [pos], c0, mask=m)

      for cp in zcopies:
        cp.wait()
      plsc.subcore_barrier()

      obase = b * C * XY
      def chunk_body(ch, carry):
        gidx = wp.at[pl.ds(ch * K, K)]
        pltpu.async_copy(feat_hbm.at[gidx], rowbuf, gsem).wait()
        cellk = [wcell[pl.ds(ch * K + t * 16, 16)] for t in range(K // 16)]
        for c in range(C):
          j = c // 2
          col0 = (c % 2) * K
          cidx = jnp.full((16,), c, jnp.int32)
          rbase = obase + c * XY
          for t in range(K // 16):
            g = plsc.load_gather(rowbuf, [widxs[t], cidx])
            colbuf[pl.ds(c * K + t * 16, 16)] = g
            didx[pl.ds(c * K + t * 16, 16)] = cellk[t] + rbase
        pltpu.async_copy(colbuf, out_hbm.at[didx], ssem).wait()
        return carry
      lax.fori_loop(0, nchunks, chunk_body, 0)

  return scatter_kernel


_scatter = _make_kernel()


@jax.jit
def kernel(input_feat, coords, batch_size):
  del batch_size
  xs = coords[..., 0].astype(jnp.int32).reshape(-1)
  ys = coords[..., 1].astype(jnp.int32).reshape(-1)
  featflat = jnp.transpose(input_feat, (0, 2, 1)).reshape(B * P, C)
  out = _scatter(featflat, xs, ys)
  return out.reshape(B, C, X, X)


# speedup vs baseline: 9.8279x; 9.8279x over previous
import functools

import jax
import jax.numpy as jnp
from jax import lax
from jax.experimental import pallas as pl
from jax.experimental.pallas import tpu as pltpu
from jax.experimental.pallas import tpu_sc as plsc

X = 512
XY = X * X
B = 4
C = 64
P = 24000
NSUB = 16
NCORE = 2
RANGE = XY // NSUB
CHK = 128
WCAP = RANGE + CHK
CCHUNK = 2000
SLAB_ROWS = P * 2
ROWS_PER_TILE = SLAB_ROWS // NSUB


def _make_kernel():
  mesh = plsc.VectorSubcoreMesh(core_axis_name="c", subcore_axis_name="s")

  @functools.partial(
      pl.kernel,
      out_type=jax.ShapeDtypeStruct((B * C * XY,), jnp.float32),
      mesh=mesh,
      compiler_params=pltpu.CompilerParams(
          needs_layout_passes=False, use_tc_tiling_on_sc=False,
          disable_bounds_checks=True, disable_semaphore_checks=True),
      scratch_types=[
          pltpu.VMEM((CCHUNK,), jnp.int32),
          pltpu.VMEM((CCHUNK,), jnp.int32),
          pltpu.VMEM((WCAP,), jnp.int32),
          pltpu.VMEM((3 * RANGE,), jnp.float32),
          pltpu.VMEM((2, CHK, 8), jnp.float32),
          pltpu.VMEM((CHK,), jnp.int32),
          pltpu.VMEM((CHK,), jnp.int32),
          pltpu.VMEM((1000, 8), jnp.float32),
          pltpu.VMEM_SHARED((SLAB_ROWS, 8), jnp.float32),
          pltpu.SemaphoreType.DMA,
          pltpu.SemaphoreType.DMA,
          pltpu.SemaphoreType.DMA,
          pltpu.SemaphoreType.DMA,
          pltpu.SemaphoreType.DMA,
      ],
  )
  def scatter_kernel(feat_hbm, xs_hbm, ys_hbm, out_hbm,
                     xc, yc, wpk, stage, rowbuf, gidx0, gidx1, slabtmp,
                     shared, gsem0, gsem1, ssem0, ssem1, ssem2):
    core = lax.axis_index("c")
    sub = lax.axis_index("s")
    lo = sub * RANGE
    iota = lax.iota(jnp.int32, 16)
    zero16 = jnp.zeros((16,), jnp.float32)
    minus1f = plsc.bitcast(jnp.full((16,), -1, jnp.int32), jnp.float32)
    widxs = [t * 16 + iota for t in range(CHK // 16)]
    ssems = [ssem0, ssem1, ssem2]

    def sz_body(i, carry):
      stage[pl.ds(i * 16, 16)] = zero16
      return carry
    lax.fori_loop(0, 3 * RANGE // 16, sz_body, 0)

    def sync_gather(ch, gl):
      for t in range(CHK // 16):
        wv = wpk[pl.ds(ch * CHK + t * 16, 16)]
        gidx0[pl.ds(t * 16, 16)] = (wv >> 14) * 2 + (gl >> 2)
      pltpu.async_copy(shared.at[gidx0], rowbuf.at[0], gsem0).wait()

    def drain_stage(slot):
      for s in range(3):
        @pl.when(slot == s)
        def _():
          pltpu.make_async_copy(
              stage.at[pl.ds(0, RANGE)],
              out_hbm.at[pl.ds(0, RANGE)], ssems[s]).wait()

    for bl in range(B // NCORE):
      b = core * (B // NCORE) + bl
      obase = b * C * XY + lo

      def mi_body(i, carry):
        stage[pl.ds(i * 16, 16)] = minus1f
        return carry
      lax.fori_loop(0, RANGE // 16, mi_body, 0)

      for cc in range(P // CCHUNK):
        pltpu.sync_copy(xs_hbm.at[pl.ds(b * P + cc * CCHUNK, CCHUNK)], xc)
        pltpu.sync_copy(ys_hbm.at[pl.ds(b * P + cc * CCHUNK, CCHUNK)], yc)
        pchunk = cc * CCHUNK

        def scan_body(i, carry):
          xv = xc[pl.ds(i * 16, 16)]
          yv = yc[pl.ds(i * 16, 16)]
          rel = (yv * X + xv) - lo
          m = (rel >= 0) & (rel < RANGE)
          pv = (pchunk + i * 16) + iota
          plsc.store_scatter(stage, [rel], plsc.bitcast(pv, jnp.float32),
                             mask=m)
          return carry
        lax.fori_loop(0, CCHUNK // 16, scan_body, 0)

      def comp_body(j, cnt):
        mv = plsc.bitcast(stage[pl.ds(j * 16, 16)], jnp.int32)
        m = mv >= 0
        pk = mv * 16384 + (j * 16 + iota)
        plsc.store_compressed(wpk.at[pl.ds(cnt, 16)], pk, mask=m)
        return cnt + jnp.max(plsc.all_reduce_population_count(m))
      count = lax.fori_loop(0, RANGE // 16, comp_body, jnp.int32(0))
      nchunks = (count + (CHK - 1)) // CHK

      @pl.when(count > 0)
      def _():
        w0 = plsc.load_gather(wpk, [jnp.zeros((16,), jnp.int32)])
        base = (nchunks - 1) * CHK
        for t in range(CHK // 16):
          pos = base + t * 16 + iota
          plsc.store_scatter(wpk, [pos], w0, mask=pos >= count)

      def z0_body(i, carry):
        stage[pl.ds(i * 16, 16)] = zero16
        return carry
      lax.fori_loop(0, RANGE // 16, z0_body, 0)

      for q in range(4):
        plsc.subcore_barrier()
        srow0 = (b * 4 + q) * SLAB_ROWS + sub * ROWS_PER_TILE
        for k in range(ROWS_PER_TILE // 1000):
          pltpu.sync_copy(feat_hbm.at[pl.ds(srow0 + k * 1000, 1000)],
                          slabtmp)
          pltpu.sync_copy(
              slabtmp,
              shared.at[pl.ds(sub * ROWS_PER_TILE + k * 1000, 1000)])
        plsc.subcore_barrier()

        def pass_body(gl, carry):
          c0 = q * 16 + 2 * gl
          slot0 = lax.rem(c0, 3)
          slot1 = lax.rem(c0 + 1, 3)

          @pl.when(c0 >= 3)
          def _():
            drain_stage(slot0)

          @pl.when(c0 + 1 >= 3)
          def _():
            drain_stage(slot1)

          @pl.when(nchunks > 0)
          def _():
            def chunk_body(ch, carry2):
              sync_gather(ch, gl)
              parv = jnp.zeros((16,), jnp.int32)
              col0 = 2 * lax.rem(gl, 4)
              for ci in range(2):
                sbase = lax.rem(c0 + ci, 3) * RANGE
                civ = jnp.zeros((16,), jnp.int32) + (col0 + ci)
                for t in range(CHK // 16):
                  wv = wpk[pl.ds(ch * CHK + t * 16, 16)]
                  rel = wv & 16383
                  val = plsc.load_gather(rowbuf, [parv, widxs[t], civ])
                  plsc.store_scatter(stage, [rel + sbase], val)
              return carry2
            lax.fori_loop(0, nchunks, chunk_body, 0)

          for ci in range(2):
            sbase = lax.rem(c0 + ci, 3) * RANGE
            dst = out_hbm.at[pl.ds(obase + (c0 + ci) * XY, RANGE)]
            for s in range(3):
              @pl.when(lax.rem(c0 + ci, 3) == s)
              def _():
                pltpu.async_copy(stage.at[pl.ds(sbase, RANGE)], dst, ssems[s])
          return carry
        lax.fori_loop(0, 8, pass_body, 0)

      for s in range(3):
        pltpu.make_async_copy(
            stage.at[pl.ds(0, RANGE)],
            out_hbm.at[pl.ds(0, RANGE)], ssems[s]).wait()

      def rz_body(ch, carry):
        for t in range(CHK // 16):
          rel = wpk[pl.ds(ch * CHK + t * 16, 16)] & 16383
          for srow in range(3):
            plsc.store_scatter(stage, [rel + srow * RANGE], zero16)
        return carry
      lax.fori_loop(0, nchunks, rz_body, 0)

  return scatter_kernel


_scatter = _make_kernel()


@jax.jit
def kernel(input_feat, coords, batch_size):
  del batch_size
  xs = coords[..., 0].astype(jnp.int32).reshape(-1)
  ys = coords[..., 1].astype(jnp.int32).reshape(-1)
  feat = jnp.transpose(input_feat, (0, 2, 1)).reshape(B, P, 4, 2, 8)
  feat = jnp.transpose(feat, (0, 2, 1, 3, 4)).reshape(B * 4 * SLAB_ROWS, 8)
  out = _scatter(feat, xs, ys)
  return out.reshape(B, C, X, X)


# speedup vs baseline: 9.9712x; 1.0146x over previous
import functools

import jax
import jax.numpy as jnp
from jax import lax
from jax.experimental import pallas as pl
from jax.experimental.pallas import tpu as pltpu
from jax.experimental.pallas import tpu_sc as plsc

X = 512
XY = X * X
B = 4
C = 64
P = 24000
NSUB = 16
NCORE = 2
RANGE = XY // NSUB
CHK = 128
SCHK = 1024
WCAP = RANGE + SCHK
CCHUNK = 2000
SLAB_ROWS = P * 2
ROWS_PER_TILE = SLAB_ROWS // NSUB


def _make_kernel():
  mesh = plsc.VectorSubcoreMesh(core_axis_name="c", subcore_axis_name="s")

  @functools.partial(
      pl.kernel,
      out_type=jax.ShapeDtypeStruct((B * C * XY,), jnp.float32),
      mesh=mesh,
      compiler_params=pltpu.CompilerParams(
          needs_layout_passes=False, use_tc_tiling_on_sc=False,
          disable_bounds_checks=True, disable_semaphore_checks=True),
      scratch_types=[
          pltpu.VMEM((CCHUNK,), jnp.int32),
          pltpu.VMEM((CCHUNK,), jnp.int32),
          pltpu.VMEM((WCAP,), jnp.int32),
          pltpu.VMEM((3 * RANGE,), jnp.float32),
          pltpu.VMEM((SCHK, 8), jnp.float32),
          pltpu.VMEM((SCHK,), jnp.int32),
          pltpu.VMEM((500, 8), jnp.float32),
          pltpu.VMEM_SHARED((SLAB_ROWS, 8), jnp.float32),
          pltpu.SemaphoreType.DMA,
          pltpu.SemaphoreType.DMA,
          pltpu.SemaphoreType.DMA,
          pltpu.SemaphoreType.DMA,
          pltpu.SemaphoreType.DMA,
      ],
  )
  def scatter_kernel(feat_hbm, xs_hbm, ys_hbm, out_hbm,
                     xc, yc, wpk, stage, rowbuf, gidx0, slabtmp,
                     shared, gsem0, gsem1, ssem0, ssem1, ssem2):
    core = lax.axis_index("c")
    sub = lax.axis_index("s")
    lo = sub * RANGE
    iota = lax.iota(jnp.int32, 16)
    zero16 = jnp.zeros((16,), jnp.float32)
    minus1f = plsc.bitcast(jnp.full((16,), -1, jnp.int32), jnp.float32)
    widxs = [t * 16 + iota for t in range(CHK // 16)]
    ssems = [ssem0, ssem1, ssem2]

    def sz_body(i, carry):
      stage[pl.ds(i * 16, 16)] = zero16
      return carry
    lax.fori_loop(0, 3 * RANGE // 16, sz_body, 0)

    def sync_gather_super(sc, gl):
      def gi_body(k, carry):
        wv = wpk[pl.ds(sc * SCHK + k * 16, 16)]
        gidx0[pl.ds(k * 16, 16)] = (wv >> 14) * 2 + (gl >> 2)
        return carry
      lax.fori_loop(0, SCHK // 16, gi_body, 0)
      pltpu.async_copy(shared.at[gidx0], rowbuf, gsem0).wait()

    def drain_stage(slot):
      for s in range(3):
        @pl.when(slot == s)
        def _():
          pltpu.make_async_copy(
              stage.at[pl.ds(0, RANGE)],
              out_hbm.at[pl.ds(0, RANGE)], ssems[s]).wait()

    for bl in range(B // NCORE):
      b = core * (B // NCORE) + bl
      obase = b * C * XY + lo

      def mi_body(i, carry):
        stage[pl.ds(i * 16, 16)] = minus1f
        return carry
      lax.fori_loop(0, RANGE // 16, mi_body, 0)

      for cc in range(P // CCHUNK):
        pltpu.sync_copy(xs_hbm.at[pl.ds(b * P + cc * CCHUNK, CCHUNK)], xc)
        pltpu.sync_copy(ys_hbm.at[pl.ds(b * P + cc * CCHUNK, CCHUNK)], yc)
        pchunk = cc * CCHUNK

        def scan_body(i, carry):
          xv = xc[pl.ds(i * 16, 16)]
          yv = yc[pl.ds(i * 16, 16)]
          rel = (yv * X + xv) - lo
          m = (rel >= 0) & (rel < RANGE)
          pv = (pchunk + i * 16) + iota
          plsc.store_scatter(stage, [rel], plsc.bitcast(pv, jnp.float32),
                             mask=m)
          return carry
        lax.fori_loop(0, CCHUNK // 16, scan_body, 0)

      def comp_body(j, cnt):
        mv = plsc.bitcast(stage[pl.ds(j * 16, 16)], jnp.int32)
        m = mv >= 0
        pk = mv * 16384 + (j * 16 + iota)
        plsc.store_compressed(wpk.at[pl.ds(cnt, 16)], pk, mask=m)
        return cnt + jnp.max(plsc.all_reduce_population_count(m))
      count = lax.fori_loop(0, RANGE // 16, comp_body, jnp.int32(0))
      nchunks = (count + (CHK - 1)) // CHK

      @pl.when(count > 0)
      def _():
        w0 = plsc.load_gather(wpk, [jnp.zeros((16,), jnp.int32)])
        padlim = ((count + (SCHK - 1)) // SCHK) * (SCHK // 16)

        def pad_body(k, carry):
          pos = k * 16 + iota
          plsc.store_scatter(wpk, [pos], w0, mask=pos >= count)
          return carry
        lax.fori_loop(count // 16, padlim, pad_body, 0)

      def z0_body(i, carry):
        stage[pl.ds(i * 16, 16)] = zero16
        return carry
      lax.fori_loop(0, RANGE // 16, z0_body, 0)

      for q in range(4):
        plsc.subcore_barrier()
        srow0 = (b * 4 + q) * SLAB_ROWS + sub * ROWS_PER_TILE
        for k in range(ROWS_PER_TILE // 500):
          pltpu.sync_copy(feat_hbm.at[pl.ds(srow0 + k * 500, 500)],
                          slabtmp)
          pltpu.sync_copy(
              slabtmp,
              shared.at[pl.ds(sub * ROWS_PER_TILE + k * 500, 500)])
        plsc.subcore_barrier()

        def pass_body(gl, carry):
          c0 = q * 16 + 2 * gl
          slot0 = lax.rem(c0, 3)
          slot1 = lax.rem(c0 + 1, 3)

          @pl.when(c0 >= 3)
          def _():
            drain_stage(slot0)

          @pl.when(c0 + 1 >= 3)
          def _():
            drain_stage(slot1)

          @pl.when(nchunks > 0)
          def _():
            def chunk_body(ch, carry2):
              @pl.when(lax.rem(ch, SCHK // CHK) == 0)
              def _():
                sync_gather_super(ch // (SCHK // CHK), gl)
              wbase = lax.rem(ch, SCHK // CHK) * CHK
              col0 = 2 * lax.rem(gl, 4)
              for ci in range(2):
                sbase = lax.rem(c0 + ci, 3) * RANGE
                civ = jnp.zeros((16,), jnp.int32) + (col0 + ci)
                for t in range(CHK // 16):
                  wv = wpk[pl.ds(ch * CHK + t * 16, 16)]
                  rel = wv & 16383
                  val = plsc.load_gather(rowbuf, [wbase + widxs[t], civ])
                  plsc.store_scatter(stage, [rel + sbase], val)
              return carry2
            lax.fori_loop(0, nchunks, chunk_body, 0)

          for ci in range(2):
            sbase = lax.rem(c0 + ci, 3) * RANGE
            dst = out_hbm.at[pl.ds(obase + (c0 + ci) * XY, RANGE)]
            for s in range(3):
              @pl.when(lax.rem(c0 + ci, 3) == s)
              def _():
                pltpu.async_copy(stage.at[pl.ds(sbase, RANGE)], dst, ssems[s])
          return carry
        lax.fori_loop(0, 8, pass_body, 0)

      for s in range(3):
        pltpu.make_async_copy(
            stage.at[pl.ds(0, RANGE)],
            out_hbm.at[pl.ds(0, RANGE)], ssems[s]).wait()

      def rz_body(ch, carry):
        for t in range(CHK // 16):
          rel = wpk[pl.ds(ch * CHK + t * 16, 16)] & 16383
          for srow in range(3):
            plsc.store_scatter(stage, [rel + srow * RANGE], zero16)
        return carry
      lax.fori_loop(0, nchunks, rz_body, 0)

  return scatter_kernel


_scatter = _make_kernel()


@jax.jit
def kernel(input_feat, coords, batch_size):
  del batch_size
  xs = coords[..., 0].astype(jnp.int32).reshape(-1)
  ys = coords[..., 1].astype(jnp.int32).reshape(-1)
  feat = jnp.transpose(input_feat, (0, 2, 1)).reshape(B, P, 4, 2, 8)
  feat = jnp.transpose(feat, (0, 2, 1, 3, 4)).reshape(B * 4 * SLAB_ROWS, 8)
  out = _scatter(feat, xs, ys)
  return out.reshape(B, C, X, X)
